# Initial kernel scaffold; baseline (speedup 1.0000x reference)
import jax
import jax.numpy as jnp
from jax.experimental import pallas as pl


def kernel(item, category, user, W_item, W_category, W_user):
    raise NotImplementedError("write your pallas kernel here")


# speedup vs baseline: 4.7558x; 4.7558x over previous
import functools

import jax
import jax.numpy as jnp
from jax import lax
from jax.experimental import pallas as pl
from jax.experimental.pallas import tpu as pltpu
from jax.experimental.pallas import tpu_sc as plsc

B, L = 4096, 50
N = B * L
D_ITEM, D_CAT, D_USER = 64, 32, 64
D_OUT = D_ITEM + D_CAT + D_USER
NUM_CORES = 2
NUM_SUBCORES = 16
NW = NUM_CORES * NUM_SUBCORES
PER_W = N // NW
CHUNK = 128
NCH = PER_W // CHUNK
LANES = 16


def _fixup_pad_rows(idx_ref, off, rows_ref, d):
    zeros = jnp.zeros((LANES,), jnp.float32)
    for g in range(CHUNK // LANES):
        iv = idx_ref[pl.ds(off + g * LANES, LANES)]
        m = iv == 0
        mn = jnp.min(iv)
        rowids = lax.iota(jnp.int32, LANES) + g * LANES

        @pl.when(mn == 0)
        def _():
            for c in range(d):
                colids = jnp.full((LANES,), c, jnp.int32)
                plsc.store_scatter(rows_ref, [rowids, colids], zeros, mask=m)


def _body(item_h, cat_h, user_h, wi_h, wc_h, wu_h, out_h,
          idxi, idxc, idxu, ri, rc, ru, sem):
    wid = lax.axis_index("s") * NUM_CORES + lax.axis_index("c")
    base = wid * PER_W

    pltpu.sync_copy(item_h.at[pl.ds(base, PER_W)], idxi)
    pltpu.sync_copy(cat_h.at[pl.ds(base, PER_W)], idxc)
    pltpu.sync_copy(user_h.at[pl.ds(base, PER_W)], idxu)

    def chunk_body(j, carry):
        off = j * CHUNK
        cp1 = pltpu.async_copy(wi_h.at[idxi.at[pl.ds(off, CHUNK)]], ri, sem)
        cp2 = pltpu.async_copy(wc_h.at[idxc.at[pl.ds(off, CHUNK)]], rc, sem)
        cp3 = pltpu.async_copy(wu_h.at[idxu.at[pl.ds(off, CHUNK)]], ru, sem)
        cp1.wait()
        cp2.wait()
        cp3.wait()

        _fixup_pad_rows(idxi, off, ri, D_ITEM)
        _fixup_pad_rows(idxc, off, rc, D_CAT)
        _fixup_pad_rows(idxu, off, ru, D_USER)

        row0 = base + off
        pltpu.sync_copy(ri, out_h.at[pl.ds(row0, CHUNK), pl.ds(0, D_ITEM)])
        pltpu.sync_copy(rc, out_h.at[pl.ds(row0, CHUNK),
                                     pl.ds(D_ITEM, D_CAT)])
        pltpu.sync_copy(ru, out_h.at[pl.ds(row0, CHUNK),
                                     pl.ds(D_ITEM + D_CAT, D_USER)])
        return carry

    lax.fori_loop(0, NCH, chunk_body, 0)


@jax.jit
def _run(item_f, cat_f, user_f, W_item, W_category, W_user):
    mesh = plsc.VectorSubcoreMesh(core_axis_name="c", subcore_axis_name="s")
    k = functools.partial(
        pl.kernel,
        mesh=mesh,
        compiler_params=pltpu.CompilerParams(
            use_tc_tiling_on_sc=False, needs_layout_passes=False),
        out_type=jax.ShapeDtypeStruct((N, D_OUT), jnp.float32),
        scratch_types=[
            pltpu.VMEM((PER_W,), jnp.int32),
            pltpu.VMEM((PER_W,), jnp.int32),
            pltpu.VMEM((PER_W,), jnp.int32),
            pltpu.VMEM((CHUNK, D_ITEM), jnp.float32),
            pltpu.VMEM((CHUNK, D_CAT), jnp.float32),
            pltpu.VMEM((CHUNK, D_USER), jnp.float32),
            pltpu.SemaphoreType.DMA,
        ],
    )(_body)
    return k(item_f, cat_f, user_f, W_item, W_category, W_user)


def kernel(item, category, user, W_item, W_category, W_user):
    item_f = item.reshape(N).astype(jnp.int32)
    cat_f = category.reshape(N).astype(jnp.int32)
    user_f = user.reshape(N).astype(jnp.int32)
    out = _run(item_f, cat_f, user_f, W_item, W_category, W_user)
    return out.reshape(B, L, D_OUT)


# speedup vs baseline: 5.3527x; 1.1255x over previous
import functools

import jax
import jax.numpy as jnp
from jax import lax
from jax.experimental import pallas as pl
from jax.experimental.pallas import tpu as pltpu
from jax.experimental.pallas import tpu_sc as plsc

B, L = 4096, 50
N = B * L
D_ITEM, D_CAT, D_USER = 64, 32, 64
D_OUT = D_ITEM + D_CAT + D_USER
NUM_CORES = 2
NUM_SUBCORES = 16
NW = NUM_CORES * NUM_SUBCORES
PER_W = N // NW
CHUNK = 128
NCH = PER_W // CHUNK
NBUF = 2
LANES = 16


def _fixup_pad_rows(idx_ref, off, rows_ref, d):
    zeros = jnp.zeros((LANES,), jnp.float32)
    for g in range(CHUNK // LANES):
        iv = idx_ref[pl.ds(off + g * LANES, LANES)]
        m = iv == 0
        mn = jnp.min(iv)
        rowids = lax.iota(jnp.int32, LANES) + g * LANES

        @pl.when(mn == 0)
        def _():
            for c in range(d):
                colids = jnp.full((LANES,), c, jnp.int32)
                plsc.store_scatter(rows_ref, [rowids, colids], zeros, mask=m)


def _body(item_h, cat_h, user_h, wi_h, wc_h, wu_h, out_h,
          idxi, idxc, idxu,
          ri0, rc0, ru0, ri1, rc1, ru1,
          gsem0, gsem1, wsem0, wsem1):
    wid = lax.axis_index("s") * NUM_CORES + lax.axis_index("c")
    base = wid * PER_W
    rows = ((ri0, rc0, ru0), (ri1, rc1, ru1))
    gsems = (gsem0, gsem1)
    wsems = (wsem0, wsem1)

    pltpu.sync_copy(item_h.at[pl.ds(base, PER_W)], idxi)
    pltpu.sync_copy(cat_h.at[pl.ds(base, PER_W)], idxc)
    pltpu.sync_copy(user_h.at[pl.ds(base, PER_W)], idxu)

    def gather_copies(j, s):
        off = j * CHUNK
        ri, rc, ru = rows[s]
        sem = gsems[s]
        return (
            pltpu.make_async_copy(wi_h.at[idxi.at[pl.ds(off, CHUNK)]],
                                  ri, sem),
            pltpu.make_async_copy(wc_h.at[idxc.at[pl.ds(off, CHUNK)]],
                                  rc, sem),
            pltpu.make_async_copy(wu_h.at[idxu.at[pl.ds(off, CHUNK)]],
                                  ru, sem),
        )

    def write_copies(j, s):
        row0 = base + j * CHUNK
        ri, rc, ru = rows[s]
        sem = wsems[s]
        return (
            pltpu.make_async_copy(
                ri, out_h.at[pl.ds(row0, CHUNK), pl.ds(0, D_ITEM)], sem),
            pltpu.make_async_copy(
                rc, out_h.at[pl.ds(row0, CHUNK), pl.ds(D_ITEM, D_CAT)], sem),
            pltpu.make_async_copy(
                ru, out_h.at[pl.ds(row0, CHUNK),
                             pl.ds(D_ITEM + D_CAT, D_USER)], sem),
        )

    for cp in gather_copies(0, 0):
        cp.start()

    def step(j, s):
        other = 1 - s

        @pl.when(jnp.logical_and(j >= 1, j + 1 < NCH))
        def _():
            for cp in write_copies(j - 1, other):
                cp.wait()

        @pl.when(j + 1 < NCH)
        def _():
            for cp in gather_copies(j + 1, other):
                cp.start()

        for cp in gather_copies(j, s):
            cp.wait()

        off = j * CHUNK
        ri, rc, ru = rows[s]
        _fixup_pad_rows(idxi, off, ri, D_ITEM)
        _fixup_pad_rows(idxc, off, rc, D_CAT)
        _fixup_pad_rows(idxu, off, ru, D_USER)

        for cp in write_copies(j, s):
            cp.start()

    def outer_body(jo, carry):
        for b in range(NBUF):
            step(jo + b, b)
        return carry

    lax.fori_loop(0, NCH // NBUF, lambda t, c: outer_body(t * NBUF, c), 0)

    for cp in write_copies(NCH - 2, (NCH - 2) % NBUF):
        cp.wait()
    for cp in write_copies(NCH - 1, (NCH - 1) % NBUF):
        cp.wait()


@jax.jit
def _run(item_f, cat_f, user_f, W_item, W_category, W_user):
    mesh = plsc.VectorSubcoreMesh(core_axis_name="c", subcore_axis_name="s")
    k = functools.partial(
        pl.kernel,
        mesh=mesh,
        compiler_params=pltpu.CompilerParams(
            use_tc_tiling_on_sc=False, needs_layout_passes=False),
        out_type=jax.ShapeDtypeStruct((N, D_OUT), jnp.float32),
        scratch_types=[
            pltpu.VMEM((PER_W,), jnp.int32),
            pltpu.VMEM((PER_W,), jnp.int32),
            pltpu.VMEM((PER_W,), jnp.int32),
            pltpu.VMEM((CHUNK, D_ITEM), jnp.float32),
            pltpu.VMEM((CHUNK, D_CAT), jnp.float32),
            pltpu.VMEM((CHUNK, D_USER), jnp.float32),
            pltpu.VMEM((CHUNK, D_ITEM), jnp.float32),
            pltpu.VMEM((CHUNK, D_CAT), jnp.float32),
            pltpu.VMEM((CHUNK, D_USER), jnp.float32),
            pltpu.SemaphoreType.DMA,
            pltpu.SemaphoreType.DMA,
            pltpu.SemaphoreType.DMA,
            pltpu.SemaphoreType.DMA,
        ],
    )(_body)
    return k(item_f, cat_f, user_f, W_item, W_category, W_user)


def kernel(item, category, user, W_item, W_category, W_user):
    item_f = item.reshape(N).astype(jnp.int32)
    cat_f = category.reshape(N).astype(jnp.int32)
    user_f = user.reshape(N).astype(jnp.int32)
    out = _run(item_f, cat_f, user_f, W_item, W_category, W_user)
    return out.reshape(B, L, D_OUT)
